# Initial kernel scaffold; baseline (speedup 1.0000x reference)
import jax
import jax.numpy as jnp
from jax.experimental import pallas as pl


def kernel(x, edge_index, reranker_scores, Wp, bp, Wl0, bl0, Wr0, Wl1, bl1, Wr1, Ws1, bs1, Ws2, bs2, alpha_logit):
    raise NotImplementedError("write your pallas kernel here")


# speedup vs baseline: 5.0232x; 5.0232x over previous
import functools

import jax
import jax.numpy as jnp
from jax import lax
from jax.experimental import pallas as pl
from jax.experimental.pallas import tpu as pltpu
from jax.experimental.pallas import tpu_sc as plsc

N = 10000
E = 320000
D = 128
H = 128

NC = 2
NS = 16
NW = NC * NS
CHUNK = 128
EPT = E // NW
NFULL = EPT // CHUNK
TAIL = EPT - NFULL * CHUNK
NP_ = 10240
ROWS_PER_SUB = NP_ // NS
ZROWS = 80

_MM = functools.partial(jnp.dot, precision=lax.Precision.HIGHEST,
                        preferred_element_type=jnp.float32)

_MESH = dict(core_axis_name="c", subcore_axis_name="s")


@functools.lru_cache(maxsize=None)
def _make_sc_agg():
    mesh = plsc.VectorSubcoreMesh(num_cores=NC, num_subcores=NS, **_MESH)
    scratch = [
        pltpu.VMEM((ZROWS, D), jnp.float32),
        pltpu.VMEM((CHUNK,), jnp.int32),
        pltpu.VMEM((CHUNK,), jnp.int32),
        pltpu.VMEM((CHUNK, D), jnp.float32),
        pltpu.VMEM((TAIL,), jnp.int32),
        pltpu.VMEM((TAIL,), jnp.int32),
        pltpu.VMEM((TAIL, D), jnp.float32),
        pltpu.VMEM_SHARED((NP_, D), jnp.float32),
        pltpu.SemaphoreType.DMA,
    ]

    @functools.partial(
        pl.kernel, mesh=mesh, scratch_types=scratch,
        out_type=jax.ShapeDtypeStruct((NC, NP_, D), jnp.float32))
    def sc_agg(h_hbm, srce_hbm, dste_hbm, zd_hbm, agg_out,
               zbuf, src_v, dst_v, rows_v, src_t, dst_t, rows_t, acc_sh, sem):
        cid = lax.axis_index("c")
        sid = lax.axis_index("s")
        wid = sid * NC + cid
        row0 = sid * ROWS_PER_SUB
        ebase = wid * EPT

        pltpu.sync_copy(zd_hbm, zbuf)

        @pl.loop(0, ROWS_PER_SUB // ZROWS)
        def _(i):
            pltpu.sync_copy(zbuf, acc_sh.at[pl.ds(row0 + i * ZROWS, ZROWS)])

        plsc.subcore_barrier()

        @pl.loop(0, NFULL)
        def _(i):
            base = ebase + i * CHUNK
            pltpu.sync_copy(srce_hbm.at[pl.ds(base, CHUNK)], src_v)
            pltpu.sync_copy(dste_hbm.at[pl.ds(base, CHUNK)], dst_v)
            pltpu.async_copy(h_hbm.at[src_v], rows_v, sem).wait()
            pltpu.sync_copy(rows_v, acc_sh.at[dst_v], add=True)

        tbase = ebase + NFULL * CHUNK
        pltpu.sync_copy(srce_hbm.at[pl.ds(tbase, TAIL)], src_t)
        pltpu.sync_copy(dste_hbm.at[pl.ds(tbase, TAIL)], dst_t)
        pltpu.async_copy(h_hbm.at[src_t], rows_t, sem).wait()
        pltpu.sync_copy(rows_t, acc_sh.at[dst_t], add=True)

        plsc.subcore_barrier()
        rows = pl.ds(row0, ROWS_PER_SUB)
        pltpu.sync_copy(acc_sh.at[rows], agg_out.at[cid, rows])

    return sc_agg


@functools.lru_cache(maxsize=None)
def _make_sc_cnt():
    mesh = plsc.VectorSubcoreMesh(num_cores=NC, num_subcores=NS, **_MESH)
    scratch = [
        pltpu.VMEM((ZROWS, D), jnp.float32),
        pltpu.VMEM((CHUNK, D), jnp.float32),
        pltpu.VMEM((CHUNK,), jnp.int32),
        pltpu.VMEM((TAIL,), jnp.int32),
        pltpu.VMEM_SHARED((NP_, D), jnp.float32),
    ]

    @functools.partial(
        pl.kernel, mesh=mesh, scratch_types=scratch,
        out_type=jax.ShapeDtypeStruct((NC, NP_, D), jnp.float32))
    def sc_cnt(dste_hbm, zd_hbm, ones_hbm, cnt_out,
               zbuf, ones_v, dst_v, dst_t, cnt_sh):
        cid = lax.axis_index("c")
        sid = lax.axis_index("s")
        wid = sid * NC + cid
        row0 = sid * ROWS_PER_SUB
        ebase = wid * EPT

        pltpu.sync_copy(zd_hbm, zbuf)
        pltpu.sync_copy(ones_hbm, ones_v)

        @pl.loop(0, ROWS_PER_SUB // ZROWS)
        def _(i):
            pltpu.sync_copy(zbuf, cnt_sh.at[pl.ds(row0 + i * ZROWS, ZROWS)])

        plsc.subcore_barrier()

        @pl.loop(0, NFULL)
        def _(i):
            base = ebase + i * CHUNK
            pltpu.sync_copy(dste_hbm.at[pl.ds(base, CHUNK)], dst_v)
            pltpu.sync_copy(ones_v, cnt_sh.at[dst_v], add=True)

        tbase = ebase + NFULL * CHUNK
        pltpu.sync_copy(dste_hbm.at[pl.ds(tbase, TAIL)], dst_t)
        pltpu.sync_copy(ones_v.at[pl.ds(0, TAIL)], cnt_sh.at[dst_t], add=True)

        plsc.subcore_barrier()
        rows = pl.ds(row0, ROWS_PER_SUB)
        pltpu.sync_copy(cnt_sh.at[rows], cnt_out.at[cid, rows])

    return sc_cnt


def _sc_agg(h, esrc, edst, zd):
    return _make_sc_agg()(h, esrc, edst, zd)


def _sc_cnt(edst, zd, ones):
    return _make_sc_cnt()(edst, zd, ones)


ROWS = 1000
_GRID = N // ROWS


def _rows_spec(minor):
    return pl.BlockSpec((ROWS, minor), lambda i: (i, 0))


def _full_spec(shape):
    nd = len(shape)
    return pl.BlockSpec(shape, lambda i, _nd=nd: (0,) * _nd)


def _norm_res_body(x_ref, wp_ref, bp_ref, xn_ref, res_ref):
    x = x_ref[...]
    nrm = jnp.sqrt(jnp.sum(x * x, axis=1, keepdims=True))
    xn = x / jnp.maximum(nrm, 1e-12)
    xn_ref[...] = xn
    res_ref[...] = _MM(xn, wp_ref[...]) + bp_ref[...]


_norm_res = pl.pallas_call(
    _norm_res_body,
    grid=(_GRID,),
    in_specs=[_rows_spec(D), _full_spec((D, H)), _full_spec((1, H))],
    out_specs=[_rows_spec(D), _rows_spec(H)],
    out_shape=[jax.ShapeDtypeStruct((N, D), jnp.float32),
               jax.ShapeDtypeStruct((N, H), jnp.float32)],
)


def _layer_body(a_ref, c_ref, h_ref, res_ref, wl_ref, bl_ref, wr_ref, out_ref):
    agg = a_ref[0] + a_ref[1]
    cnt = c_ref[0, :, 0:1] + c_ref[1, :, 0:1]
    mean = agg / jnp.maximum(cnt, 1.0)
    h = h_ref[...]
    pre = _MM(mean, wl_ref[...]) + bl_ref[...] + _MM(h, wr_ref[...])
    out_ref[...] = jnp.maximum(pre, 0.0) + res_ref[...]


def _agg_spec():
    return pl.BlockSpec((NC, ROWS, D), lambda i: (0, i, 0))


_sage_layer = pl.pallas_call(
    _layer_body,
    grid=(_GRID,),
    in_specs=[_agg_spec(), _agg_spec(), _rows_spec(D), _rows_spec(H),
              _full_spec((D, H)), _full_spec((1, H)), _full_spec((D, H))],
    out_specs=_rows_spec(H),
    out_shape=jax.ShapeDtypeStruct((N, H), jnp.float32),
)


def _head_body(h2_ref, rr_ref, ws1_ref, bs1_ref, ws2_ref, bs2_ref,
               alpha_ref, out_ref):
    h2 = h2_ref[...]
    t = jnp.maximum(_MM(h2, ws1_ref[...]) + bs1_ref[...], 0.0)
    g = jnp.sum(t * ws2_ref[...], axis=1, keepdims=True) + bs2_ref[...]
    alpha = alpha_ref[...]
    out_ref[...] = alpha * rr_ref[...] + (1.0 - alpha) * g


_head = pl.pallas_call(
    _head_body,
    grid=(_GRID,),
    in_specs=[_rows_spec(H), _rows_spec(1), _full_spec((H, H // 2)),
              _full_spec((1, H // 2)), _full_spec((1, H // 2)),
              _full_spec((1, 1)), _full_spec((1, 1))],
    out_specs=_rows_spec(1),
    out_shape=jax.ShapeDtypeStruct((N, 1), jnp.float32),
)


def kernel(x, edge_index, reranker_scores, Wp, bp, Wl0, bl0, Wr0, Wl1, bl1,
           Wr1, Ws1, bs1, Ws2, bs2, alpha_logit):
    zd = jnp.zeros((ZROWS, D), jnp.float32)
    ones = jnp.ones((CHUNK, D), jnp.float32)
    esrc = edge_index[0]
    edst = edge_index[1]

    cnt = _sc_cnt(edst, zd, ones)
    xn, res = _norm_res(x, Wp, bp.reshape(1, H))
    agg0 = _sc_agg(xn, esrc, edst, zd)
    h1 = _sage_layer(agg0, cnt, xn, res, Wl0, bl0.reshape(1, H), Wr0)
    agg1 = _sc_agg(h1, esrc, edst, zd)
    h2 = _sage_layer(agg1, cnt, h1, h1, Wl1, bl1.reshape(1, H), Wr1)

    alpha = jax.nn.sigmoid(alpha_logit).reshape(1, 1)
    out = _head(h2, reranker_scores.reshape(N, 1), Ws1, bs1.reshape(1, H // 2),
                Ws2.reshape(1, H // 2), bs2.reshape(1, 1), alpha)
    return out[:, 0]
